# baseline (speedup 1.0000x reference)
import jax
import jax.numpy as jnp
from jax.experimental import pallas as pl
from jax.experimental.pallas import tpu as pltpu

NPRI = 16800
THRESH = 0.35
NEGPOS = 7
F32 = jnp.float32
BF16 = jnp.bfloat16


def _smooth_l1(d):
    ad = jnp.abs(d)
    return jnp.where(ad < 1.0, 0.5 * d * d, ad - 0.5)


def _match_kernel(pc_ref, off_ref, rsc_ref, targets_ref, loc_ref, landm_ref,
                  conf_ref, key_o, conft_o, clazz_o, cdt_o, part_o):
    i = pl.program_id(0)
    tg = targets_ref[0]
    T = tg.shape[0]
    px0 = pc_ref[0:1, :]
    py0 = pc_ref[1:2, :]
    px1 = pc_ref[2:3, :]
    py1 = pc_ref[3:4, :]
    area_p = pc_ref[4:5, :]
    pcx = pc_ref[5:6, :]
    pcy = pc_ref[6:7, :]
    rpw10 = pc_ref[7:8, :]
    rph10 = pc_ref[8:9, :]
    rpw = pc_ref[9:10, :]
    rph = pc_ref[10:11, :]

    tx0 = tg[:, 0:1]
    ty0 = tg[:, 1:2]
    tx1 = tg[:, 2:3]
    ty1 = tg[:, 3:4]
    iw = jnp.maximum(jnp.minimum(px1, tx1) - jnp.maximum(px0, tx0), 0.0)
    ih = jnp.maximum(jnp.minimum(py1, ty1) - jnp.maximum(py0, ty0), 0.0)
    inter = iw * ih
    area_t = (tx1 - tx0) * (ty1 - ty0)
    ov = inter / (area_t + area_p - inter)

    ti = jax.lax.broadcasted_iota(jnp.int32, (T, 1), 0)
    pid = jax.lax.broadcasted_iota(jnp.int32, (1, NPRI), 1)

    bto = jnp.max(ov, axis=0, keepdims=True)
    bti = jnp.min(jnp.where(ov == bto, ti, T), axis=0, keepdims=True)
    bpo = jnp.max(ov, axis=1, keepdims=True)
    bp = jnp.min(jnp.where(ov == bpo, pid, NPRI), axis=1, keepdims=True)

    ovr = jnp.max(jnp.where(pid == bp, ti, -1), axis=0, keepdims=True)
    has = ovr >= 0
    bti2 = jnp.where(has, ovr, bti)
    bto2 = jnp.where(has, 2.0, bto)

    conf_t = jnp.where(bto2 < THRESH, 0.0, 1.0)
    pos = conf_t > 0.0

    onehot = (ti == bti2).astype(BF16)
    tg_hi = tg.astype(BF16)
    tg_lo = (tg - tg_hi.astype(F32)).astype(BF16)
    dn = (((0,), (0,)), ((), ()))
    matches = (
        jax.lax.dot_general(tg_hi, onehot, dn, preferred_element_type=F32)
        + jax.lax.dot_general(tg_lo, onehot, dn, preferred_element_type=F32))

    clazz = jnp.where(bto2 < THRESH, 0.0, matches[40:41, :])

    mx0 = matches[0:1]
    my0 = matches[1:2]
    mx1 = matches[2:3]
    my1 = matches[3:4]
    g_cx = ((mx0 + mx1) * 0.5 - pcx) * rpw10
    g_cy = ((my0 + my1) * 0.5 - pcy) * rph10
    g_w = jnp.log((mx1 - mx0) * rpw) * 5.0
    g_h = jnp.log((my1 - my0) * rph) * 5.0
    loc_t = jnp.concatenate([g_cx, g_cy, g_w, g_h], axis=0)
    sl_loc = _smooth_l1(loc_ref[0].astype(F32) - loc_t)
    loss_l_i = jnp.sum(jnp.where(pos, sl_loc, 0.0))

    m36 = matches[4:40]
    lm_t = (m36 - off_ref[...]) * rsc_ref[...]
    xt36 = jnp.transpose(landm_ref[0], (1, 0))
    sl_lm = _smooth_l1(xt36 - lm_t)
    loss_lm_i = jnp.sum(jnp.where(pos, sl_lm, 0.0))

    conf_row = jnp.transpose(conf_ref[0], (1, 0))
    lc = jnp.where(pos, 0.0, conf_row - conf_t)
    bbits = jax.lax.bitcast_convert_type(lc, jnp.int32)
    key = jnp.where(bbits < 0, bbits ^ jnp.int32(0x7FFFFFFF), bbits)

    key_o[pl.ds(i, 1), :] = key
    conft_o[pl.ds(i, 1), :] = conf_t
    clazz_o[pl.ds(i, 1), :] = clazz
    cdt_o[pl.ds(i, 1), :] = conf_row
    lane = jax.lax.broadcasted_iota(jnp.int32, (1, 128), 1)
    part_o[pl.ds(i, 1), :] = jnp.where(lane == 0, loss_l_i,
                                       jnp.where(lane == 1, loss_lm_i, 0.0))


def _mine_kernel(key_ref, conft_ref, clazz_ref, cdt_ref, class_ref, part_ref,
                 out_ref):
    K = key_ref[...]
    CT = conft_ref[...]
    CZ = clazz_ref[...]
    CD = cdt_ref[...]
    N = K.shape[0]
    np_f = jnp.sum(CT, axis=1, keepdims=True)
    kk = jnp.minimum(NEGPOS * np_f.astype(jnp.int32), NPRI - 1)
    kkf = kk.astype(F32)

    h = jnp.right_shift(K, 16)
    lw = jnp.bitwise_and(K, jnp.int32(0xFFFF))

    def hbody(_, c):
        lo, hi = c
        mid = jnp.right_shift(lo + hi, 1)
        cnt = jnp.sum((h >= mid).astype(F32), axis=1, keepdims=True)
        good = cnt >= kkf
        return (jnp.where(good, mid, lo), jnp.where(good, hi, mid))

    lo0 = jnp.full((N, 1), -32768, jnp.int32)
    hi0 = jnp.full((N, 1), 32768, jnp.int32)
    hstar, _ = jax.lax.fori_loop(0, 16, hbody, (lo0, hi0))

    hgt = h > hstar
    heq = h == hstar
    cgt_h = jnp.sum(hgt.astype(F32), axis=1, keepdims=True)
    k2f = kkf - cgt_h
    lmsk = jnp.where(heq, lw, -1)

    def lbody(_, c):
        lo, hi = c
        mid = jnp.right_shift(lo + hi, 1)
        cnt = jnp.sum((lmsk >= mid).astype(F32), axis=1, keepdims=True)
        good = cnt >= k2f
        return (jnp.where(good, mid, lo), jnp.where(good, hi, mid))

    lo1 = jnp.zeros((N, 1), jnp.int32)
    hi1 = jnp.full((N, 1), 65536, jnp.int32)
    lstar, _ = jax.lax.fori_loop(0, 16, lbody, (lo1, hi1))

    gt = hgt | (heq & (lw > lstar))
    eq = heq & (lw == lstar)
    rrf = kkf - jnp.sum(gt.astype(F32), axis=1, keepdims=True)

    pid2 = jax.lax.broadcasted_iota(jnp.int32, (1, NPRI), 1)

    def tbody(_, c):
        lo, hi = c
        mid = jnp.right_shift(lo + hi, 1)
        cnt = jnp.sum((eq & (pid2 <= mid)).astype(F32), axis=1, keepdims=True)
        good = cnt >= rrf
        return (jnp.where(good, lo, mid), jnp.where(good, mid, hi))

    lo2 = jnp.full((N, 1), -1, jnp.int32)
    hi2 = jnp.full((N, 1), NPRI - 1, jnp.int32)
    _, pstar = jax.lax.fori_loop(0, 15, tbody, (lo2, hi2))

    neg = gt | (eq & (pid2 <= pstar))
    maskb = (CT > 0.0) | neg
    loss_conf = jnp.sum(jnp.where(maskb, _smooth_l1(CD - CT), 0.0))

    c0 = class_ref[:, 0, :]
    c1 = class_ref[:, 1, :]
    c2 = class_ref[:, 2, :]
    mxl = jnp.maximum(jnp.maximum(c0, c1), c2)
    lse = mxl + jnp.log(jnp.exp(c0 - mxl) + jnp.exp(c1 - mxl) + jnp.exp(c2 - mxl))
    czi = CZ.astype(jnp.int32)
    selc = jnp.where(czi <= 0, c0, jnp.where(czi == 1, c1, c2))
    loss_c = jnp.sum(jnp.where(maskb, lse - selc, 0.0))

    part = part_ref[...]
    loss_l_tot = jnp.sum(part[:, 0:1])
    loss_lm_tot = jnp.sum(part[:, 1:2])
    ntot = jnp.maximum(jnp.sum(np_f), 1.0)
    lane = jax.lax.broadcasted_iota(jnp.int32, (1, 128), 1)
    vec = jnp.where(
        lane == 0, loss_l_tot,
        jnp.where(lane == 1, loss_conf,
                  jnp.where(lane == 2, loss_lm_tot, loss_c))) / ntot
    out_ref[...] = vec


def kernel(loc_data, conf_data, landm_data, class_data, priors, targets):
    num = loc_data.shape[0]
    locT = jnp.transpose(loc_data, (0, 2, 1)).astype(BF16)
    classT = jnp.transpose(class_data, (0, 2, 1))

    pt = jnp.transpose(priors)
    pcx, pcy, pw, ph = pt[0:1], pt[1:2], pt[2:3], pt[3:4]
    rpw = 1.0 / pw
    rph = 1.0 / ph
    pc = jnp.concatenate([
        pcx - pw * 0.5, pcy - ph * 0.5, pcx + pw * 0.5, pcy + ph * 0.5,
        pw * ph, pcx, pcy, 10.0 * rpw, 10.0 * rph, rpw, rph], axis=0)
    r36 = jnp.arange(36, dtype=jnp.int32)[:, None]
    evn = (r36 % 2) == 0
    off36 = jnp.where(evn, pcx, pcy)
    rsc36 = jnp.where(evn, rpw, rph) * 10.0

    keys, conft, clazz, cdt, part = pl.pallas_call(
        _match_kernel,
        grid=(num,),
        in_specs=[
            pl.BlockSpec((11, NPRI), lambda i: (0, 0)),
            pl.BlockSpec((36, NPRI), lambda i: (0, 0)),
            pl.BlockSpec((36, NPRI), lambda i: (0, 0)),
            pl.BlockSpec((1, targets.shape[1], targets.shape[2]), lambda i: (i, 0, 0)),
            pl.BlockSpec((1, 4, NPRI), lambda i: (i, 0, 0)),
            pl.BlockSpec((1, NPRI, 36), lambda i: (i, 0, 0)),
            pl.BlockSpec((1, NPRI, 1), lambda i: (i, 0, 0)),
        ],
        out_specs=[
            pl.BlockSpec((num, NPRI), lambda i: (0, 0)),
            pl.BlockSpec((num, NPRI), lambda i: (0, 0)),
            pl.BlockSpec((num, NPRI), lambda i: (0, 0)),
            pl.BlockSpec((num, NPRI), lambda i: (0, 0)),
            pl.BlockSpec((num, 128), lambda i: (0, 0)),
        ],
        out_shape=[
            jax.ShapeDtypeStruct((num, NPRI), jnp.int32),
            jax.ShapeDtypeStruct((num, NPRI), F32),
            jax.ShapeDtypeStruct((num, NPRI), F32),
            jax.ShapeDtypeStruct((num, NPRI), F32),
            jax.ShapeDtypeStruct((num, 128), F32),
        ],
    )(pc, off36, rsc36, targets, locT, landm_data, conf_data)

    out = pl.pallas_call(
        _mine_kernel,
        grid=(1,),
        in_specs=[
            pl.BlockSpec((num, NPRI), lambda i: (0, 0)),
            pl.BlockSpec((num, NPRI), lambda i: (0, 0)),
            pl.BlockSpec((num, NPRI), lambda i: (0, 0)),
            pl.BlockSpec((num, NPRI), lambda i: (0, 0)),
            pl.BlockSpec((num, 3, NPRI), lambda i: (0, 0, 0)),
            pl.BlockSpec((num, 128), lambda i: (0, 0)),
        ],
        out_specs=pl.BlockSpec((1, 128), lambda i: (0, 0)),
        out_shape=jax.ShapeDtypeStruct((1, 128), F32),
    )(keys, conft, clazz, cdt, classT, part)
    return (out[0, 0], out[0, 1], out[0, 2], out[0, 3])


# speedup vs baseline: 1.9329x; 1.9329x over previous
import jax
import jax.numpy as jnp
from jax.experimental import pallas as pl
from jax.experimental.pallas import tpu as pltpu

NPRI = 16800
THRESH = 0.35
NEGPOS = 7
F32 = jnp.float32
BF16 = jnp.bfloat16


def _smooth_l1(d):
    ad = jnp.abs(d)
    return jnp.where(ad < 1.0, 0.5 * d * d, ad - 0.5)


def _match_kernel(pc_ref, off_ref, rsc_ref, targets_ref, loc_ref, landm_ref,
                  conf_ref, key_o, conft_o, clazz_o, part_o):
    i = pl.program_id(0)
    tg = targets_ref[0]
    T = tg.shape[0]
    px0 = pc_ref[0:1, :]
    py0 = pc_ref[1:2, :]
    px1 = pc_ref[2:3, :]
    py1 = pc_ref[3:4, :]
    area_p = pc_ref[4:5, :]
    pcx = pc_ref[5:6, :]
    pcy = pc_ref[6:7, :]
    rpw10 = pc_ref[7:8, :]
    rph10 = pc_ref[8:9, :]
    rpw = pc_ref[9:10, :]
    rph = pc_ref[10:11, :]

    tx0 = tg[:, 0:1]
    ty0 = tg[:, 1:2]
    tx1 = tg[:, 2:3]
    ty1 = tg[:, 3:4]
    iw = jnp.maximum(jnp.minimum(px1, tx1) - jnp.maximum(px0, tx0), 0.0)
    ih = jnp.maximum(jnp.minimum(py1, ty1) - jnp.maximum(py0, ty0), 0.0)
    inter = iw * ih
    area_t = (tx1 - tx0) * (ty1 - ty0)
    ov = inter / (area_t + area_p - inter)

    ti = jax.lax.broadcasted_iota(jnp.int32, (T, 1), 0)
    pid = jax.lax.broadcasted_iota(jnp.int32, (1, NPRI), 1)

    bto = jnp.max(ov, axis=0, keepdims=True)
    bti = jnp.min(jnp.where(ov == bto, ti, T), axis=0, keepdims=True)
    bpo = jnp.max(ov, axis=1, keepdims=True)
    bp = jnp.min(jnp.where(ov == bpo, pid, NPRI), axis=1, keepdims=True)

    ovr = jnp.max(jnp.where(pid == bp, ti, -1), axis=0, keepdims=True)
    has = ovr >= 0
    bti2 = jnp.where(has, ovr, bti)
    bto2 = jnp.where(has, 2.0, bto)

    conf_t = jnp.where(bto2 < THRESH, 0.0, 1.0)
    pos = conf_t > 0.0

    onehot = (ti == bti2).astype(BF16)
    tg_hi = tg.astype(BF16)
    tg_lo = (tg - tg_hi.astype(F32)).astype(BF16)
    dn = (((0,), (0,)), ((), ()))
    matches = (
        jax.lax.dot_general(tg_hi, onehot, dn, preferred_element_type=F32)
        + jax.lax.dot_general(tg_lo, onehot, dn, preferred_element_type=F32))

    clazz = jnp.where(bto2 < THRESH, 0.0, matches[40:41, :])

    mx0 = matches[0:1]
    my0 = matches[1:2]
    mx1 = matches[2:3]
    my1 = matches[3:4]
    g_cx = ((mx0 + mx1) * 0.5 - pcx) * rpw10
    g_cy = ((my0 + my1) * 0.5 - pcy) * rph10
    g_w = jnp.log((mx1 - mx0) * rpw) * 5.0
    g_h = jnp.log((my1 - my0) * rph) * 5.0
    loc_t = jnp.concatenate([g_cx, g_cy, g_w, g_h], axis=0)
    sl_loc = _smooth_l1(loc_ref[0].astype(F32) - loc_t)
    loss_l_i = jnp.sum(jnp.where(pos, sl_loc, 0.0))

    m36 = matches[4:40]
    lm_t = (m36 - off_ref[...]) * rsc_ref[...]
    sl_lm = _smooth_l1(landm_ref[0].astype(F32) - lm_t)
    loss_lm_i = jnp.sum(jnp.where(pos, sl_lm, 0.0))

    lc = jnp.where(pos, 0.0, conf_ref[pl.ds(i, 1), :] - conf_t)
    bbits = jax.lax.bitcast_convert_type(lc, jnp.int32)
    key = jnp.where(bbits < 0, bbits ^ jnp.int32(0x7FFFFFFF), bbits)

    key_o[pl.ds(i, 1), :] = key
    conft_o[pl.ds(i, 1), :] = conf_t
    clazz_o[pl.ds(i, 1), :] = clazz
    lane = jax.lax.broadcasted_iota(jnp.int32, (1, 128), 1)
    part_o[pl.ds(i, 1), :] = jnp.where(lane == 0, loss_l_i,
                                       jnp.where(lane == 1, loss_lm_i, 0.0))


def _mine_kernel(key_ref, conft_ref, clazz_ref, conf_ref, class_ref, part_ref,
                 out_ref):
    K = key_ref[...]
    CT = conft_ref[...]
    CZ = clazz_ref[...]
    CD = conf_ref[...]
    N = K.shape[0]
    np_f = jnp.sum(CT, axis=1, keepdims=True)
    kk = jnp.minimum(NEGPOS * np_f.astype(jnp.int32), NPRI - 1)
    kkf = kk.astype(F32)

    h = jnp.right_shift(K, 16)
    lw = jnp.bitwise_and(K, jnp.int32(0xFFFF))

    def hbody(_, c):
        lo, hi = c
        mid = jnp.right_shift(lo + hi, 1)
        cnt = jnp.sum((h >= mid).astype(F32), axis=1, keepdims=True)
        good = cnt >= kkf
        return (jnp.where(good, mid, lo), jnp.where(good, hi, mid))

    lo0 = jnp.full((N, 1), -32768, jnp.int32)
    hi0 = jnp.full((N, 1), 32768, jnp.int32)
    hstar, _ = jax.lax.fori_loop(0, 16, hbody, (lo0, hi0))

    hgt = h > hstar
    heq = h == hstar
    cgt_h = jnp.sum(hgt.astype(F32), axis=1, keepdims=True)
    k2f = kkf - cgt_h
    lmsk = jnp.where(heq, lw, -1)

    def lbody(_, c):
        lo, hi = c
        mid = jnp.right_shift(lo + hi, 1)
        cnt = jnp.sum((lmsk >= mid).astype(F32), axis=1, keepdims=True)
        good = cnt >= k2f
        return (jnp.where(good, mid, lo), jnp.where(good, hi, mid))

    lo1 = jnp.zeros((N, 1), jnp.int32)
    hi1 = jnp.full((N, 1), 65536, jnp.int32)
    lstar, _ = jax.lax.fori_loop(0, 16, lbody, (lo1, hi1))

    gt = hgt | (heq & (lw > lstar))
    eq = heq & (lw == lstar)
    rrf = kkf - jnp.sum(gt.astype(F32), axis=1, keepdims=True)

    pid2 = jax.lax.broadcasted_iota(jnp.int32, (1, NPRI), 1)

    def tbody(_, c):
        lo, hi = c
        mid = jnp.right_shift(lo + hi, 1)
        cnt = jnp.sum((eq & (pid2 <= mid)).astype(F32), axis=1, keepdims=True)
        good = cnt >= rrf
        return (jnp.where(good, lo, mid), jnp.where(good, mid, hi))

    lo2 = jnp.full((N, 1), -1, jnp.int32)
    hi2 = jnp.full((N, 1), NPRI - 1, jnp.int32)
    _, pstar = jax.lax.fori_loop(0, 15, tbody, (lo2, hi2))

    neg = gt | (eq & (pid2 <= pstar))
    maskb = (CT > 0.0) | neg
    loss_conf = jnp.sum(jnp.where(maskb, _smooth_l1(CD - CT), 0.0))

    c0 = class_ref[:, 0, :]
    c1 = class_ref[:, 1, :]
    c2 = class_ref[:, 2, :]
    mxl = jnp.maximum(jnp.maximum(c0, c1), c2)
    lse = mxl + jnp.log(jnp.exp(c0 - mxl) + jnp.exp(c1 - mxl) + jnp.exp(c2 - mxl))
    czi = CZ.astype(jnp.int32)
    selc = jnp.where(czi <= 0, c0, jnp.where(czi == 1, c1, c2))
    loss_c = jnp.sum(jnp.where(maskb, lse - selc, 0.0))

    part = part_ref[...]
    loss_l_tot = jnp.sum(part[:, 0:1])
    loss_lm_tot = jnp.sum(part[:, 1:2])
    ntot = jnp.maximum(jnp.sum(np_f), 1.0)
    lane = jax.lax.broadcasted_iota(jnp.int32, (1, 128), 1)
    vec = jnp.where(
        lane == 0, loss_l_tot,
        jnp.where(lane == 1, loss_conf,
                  jnp.where(lane == 2, loss_lm_tot, loss_c))) / ntot
    out_ref[...] = vec


def kernel(loc_data, conf_data, landm_data, class_data, priors, targets):
    num = loc_data.shape[0]
    locT = jnp.transpose(loc_data, (0, 2, 1)).astype(BF16)
    landmT = jnp.transpose(landm_data, (0, 2, 1)).astype(BF16)
    classT = jnp.transpose(class_data, (0, 2, 1))
    confT = conf_data[:, :, 0]

    pt = jnp.transpose(priors)
    pcx, pcy, pw, ph = pt[0:1], pt[1:2], pt[2:3], pt[3:4]
    rpw = 1.0 / pw
    rph = 1.0 / ph
    pc = jnp.concatenate([
        pcx - pw * 0.5, pcy - ph * 0.5, pcx + pw * 0.5, pcy + ph * 0.5,
        pw * ph, pcx, pcy, 10.0 * rpw, 10.0 * rph, rpw, rph], axis=0)
    r36 = jnp.arange(36, dtype=jnp.int32)[:, None]
    evn = (r36 % 2) == 0
    off36 = jnp.where(evn, pcx, pcy)
    rsc36 = jnp.where(evn, rpw, rph) * 10.0

    keys, conft, clazz, part = pl.pallas_call(
        _match_kernel,
        grid=(num,),
        in_specs=[
            pl.BlockSpec((11, NPRI), lambda i: (0, 0)),
            pl.BlockSpec((36, NPRI), lambda i: (0, 0)),
            pl.BlockSpec((36, NPRI), lambda i: (0, 0)),
            pl.BlockSpec((1, targets.shape[1], targets.shape[2]), lambda i: (i, 0, 0)),
            pl.BlockSpec((1, 4, NPRI), lambda i: (i, 0, 0)),
            pl.BlockSpec((1, 36, NPRI), lambda i: (i, 0, 0)),
            pl.BlockSpec((num, NPRI), lambda i: (0, 0)),
        ],
        out_specs=[
            pl.BlockSpec((num, NPRI), lambda i: (0, 0)),
            pl.BlockSpec((num, NPRI), lambda i: (0, 0)),
            pl.BlockSpec((num, NPRI), lambda i: (0, 0)),
            pl.BlockSpec((num, 128), lambda i: (0, 0)),
        ],
        out_shape=[
            jax.ShapeDtypeStruct((num, NPRI), jnp.int32),
            jax.ShapeDtypeStruct((num, NPRI), F32),
            jax.ShapeDtypeStruct((num, NPRI), F32),
            jax.ShapeDtypeStruct((num, 128), F32),
        ],
    )(pc, off36, rsc36, targets, locT, landmT, confT)

    out = pl.pallas_call(
        _mine_kernel,
        grid=(1,),
        in_specs=[
            pl.BlockSpec((num, NPRI), lambda i: (0, 0)),
            pl.BlockSpec((num, NPRI), lambda i: (0, 0)),
            pl.BlockSpec((num, NPRI), lambda i: (0, 0)),
            pl.BlockSpec((num, NPRI), lambda i: (0, 0)),
            pl.BlockSpec((num, 3, NPRI), lambda i: (0, 0, 0)),
            pl.BlockSpec((num, 128), lambda i: (0, 0)),
        ],
        out_specs=pl.BlockSpec((1, 128), lambda i: (0, 0)),
        out_shape=jax.ShapeDtypeStruct((1, 128), F32),
    )(keys, conft, clazz, confT, classT, part)
    return (out[0, 0], out[0, 1], out[0, 2], out[0, 3])
